# baseline (speedup 1.0000x reference)
import functools

import jax
import jax.numpy as jnp
from jax import lax
from jax.experimental import pallas as pl
from jax.experimental.pallas import tpu as pltpu
from jax.experimental.pallas import tpu_sc as plsc

B = 16384
F = 26
V = 1000000

NC, NS, L = 2, 16, 16
NW = NC * NS
RPW = B // NW
VP = 1000064


VMAIN = (V // 128) * 128
VTAIL = V - VMAIN


def _detile_body(tab_hbm, out_hbm, buf_a, buf_b, tails_a, tails_b,
                 sem_in, sem_out, sem_tail):
    def in_group(g, buf):
        return pltpu.make_async_copy(tab_hbm.at[pl.ds(8 * g, 8)], buf, sem_in)

    def in_row(f, buf, r):
        return pltpu.make_async_copy(tab_hbm.at[f], buf.at[r], sem_in)

    def outs(buf, tails, rows, base_f):
        cps = []
        for r in rows:
            f = base_f + r
            tails[r, pl.ds(0, VTAIL)] = buf[r, pl.ds(VMAIN, VTAIL)]
            cps.append(pltpu.make_async_copy(
                buf.at[r, pl.ds(0, VMAIN)],
                out_hbm.at[pl.ds(f * VP, VMAIN)], sem_out))
            cps.append(pltpu.make_async_copy(
                tails.at[r], out_hbm.at[pl.ds(f * VP + VMAIN, 128)],
                sem_tail))
        for c in cps:
            c.start()
        return cps

    cp_a = in_group(0, buf_a)
    cp_b = in_group(1, buf_b)
    cp_a.start()
    cp_b.start()

    cp_a.wait()
    outs_a = outs(buf_a, tails_a, range(8), 0)
    cp_b.wait()
    outs_b = outs(buf_b, tails_b, range(8), 8)
    for c in outs_a:
        c.wait()
    cp_a = in_group(2, buf_a)
    cp_a.start()
    cp_a.wait()
    outs_a = outs(buf_a, tails_a, range(8), 16)
    for c in outs_b:
        c.wait()
    cp_b0 = in_row(24, buf_b, 0)
    cp_b1 = in_row(25, buf_b, 1)
    cp_b0.start()
    cp_b1.start()
    cp_b0.wait()
    cp_b1.wait()
    outs_b = outs(buf_b, tails_b, range(2), 24)
    for c in outs_a:
        c.wait()
    for c in outs_b:
        c.wait()


def _detile(tables):
    return pl.pallas_call(
        _detile_body,
        in_specs=[pl.BlockSpec(memory_space=pl.ANY)],
        out_specs=pl.BlockSpec(memory_space=pl.ANY),
        out_shape=jax.ShapeDtypeStruct((F * VP,), jnp.float32),
        scratch_shapes=[
            pltpu.VMEM((8, V), jnp.float32),
            pltpu.VMEM((8, V), jnp.float32),
            pltpu.VMEM((8, 128), jnp.float32),
            pltpu.VMEM((8, 128), jnp.float32),
            pltpu.SemaphoreType.DMA,
            pltpu.SemaphoreType.DMA,
            pltpu.SemaphoreType.DMA,
        ],
        compiler_params=pltpu.CompilerParams(
            vmem_limit_bytes=100 * 1024 * 1024),
    )(tables)


EPW = RPW * F


def _body(x_hbm, tab_hbm, out_hbm, xrows, idx_v, g_v, o_v, sem):
    wid = lax.axis_index("s") * NC + lax.axis_index("c")
    base = wid * RPW

    pltpu.sync_copy(x_hbm.at[pl.ds(base, RPW)], xrows)
    lane = lax.iota(jnp.int32, L)

    def tr_body(j, _):
        rows = lane + j * L
        for f in range(F):
            idx_v[pl.ds(f * RPW + j * L, L)] = plsc.load_gather(
                xrows, [rows, jnp.full((L,), f, jnp.int32)]) + f * VP
        return 0

    lax.fori_loop(0, RPW // L, tr_body, 0)

    pltpu.async_copy(tab_hbm.at[idx_v], g_v, sem).wait()

    def red_body(j, _):
        sl = pl.ds(j * L, L)
        acc = g_v[pl.ds(j * L, L)]
        for f in range(1, F):
            acc = acc + g_v[pl.ds(f * RPW + j * L, L)]
        o_v[sl] = acc
        return 0

    lax.fori_loop(0, RPW // L, red_body, 0)

    pltpu.sync_copy(o_v, out_hbm.at[pl.ds(base, RPW)])


@jax.jit
def kernel(X, tables):
    tab_flat = _detile(tables)
    run = functools.partial(
        pl.kernel,
        out_type=jax.ShapeDtypeStruct((B,), jnp.float32),
        mesh=plsc.VectorSubcoreMesh(core_axis_name="c", subcore_axis_name="s"),
        scratch_types=[
            pltpu.VMEM((RPW, F), jnp.int32),
            pltpu.VMEM((EPW,), jnp.int32),
            pltpu.VMEM((EPW,), jnp.float32),
            pltpu.VMEM((RPW,), jnp.float32),
            pltpu.SemaphoreType.DMA,
        ],
        compiler_params=pltpu.CompilerParams(needs_layout_passes=False),
    )(_body)
    out = run(X, tab_flat)
    return out.reshape(B, 1)


# speedup vs baseline: 1.0117x; 1.0117x over previous
import functools

import jax
import jax.numpy as jnp
from jax import lax
from jax.experimental import pallas as pl
from jax.experimental.pallas import tpu as pltpu
from jax.experimental.pallas import tpu_sc as plsc

B = 16384
F = 26
V = 1000000

NC, NS, L = 2, 16, 16
NW = NC * NS
RPW = B // NW
VP = 1000064

VMAIN = (V // 128) * 128
VTAIL = V - VMAIN

CHUNKS = ((0, 16), (16, 26))


def _detile_body(f_lo, f_hi, tab_hbm, out_hbm, buf_a, buf_b, tails_a, tails_b,
                 sem_in, sem_out, sem_tail):
    nf = f_hi - f_lo
    full_groups = [(f_lo + 8 * g, 8) for g in range(nf // 8)]
    rest = nf % 8

    def in_rows(row0, nrows, buf):
        if nrows == 8:
            return [pltpu.make_async_copy(
                tab_hbm.at[pl.ds(row0, 8)], buf, sem_in)]
        return [pltpu.make_async_copy(tab_hbm.at[row0 + r], buf.at[r], sem_in)
                for r in range(nrows)]

    def outs(buf, tails, nrows, f0):
        cps = []
        for r in range(nrows):
            fo = f0 - f_lo + r
            tails[r, pl.ds(0, VTAIL)] = buf[r, pl.ds(VMAIN, VTAIL)]
            cps.append(pltpu.make_async_copy(
                buf.at[r, pl.ds(0, VMAIN)],
                out_hbm.at[pl.ds(fo * VP, VMAIN)], sem_out))
            cps.append(pltpu.make_async_copy(
                tails.at[r], out_hbm.at[pl.ds(fo * VP + VMAIN, 128)],
                sem_tail))
        for c in cps:
            c.start()
        return cps

    items = full_groups + ([(f_lo + 8 * (nf // 8), rest)] if rest else [])
    bufs = [(buf_a, tails_a), (buf_b, tails_b)]
    inflight = {}
    pending_outs = {}
    for i, (row0, nrows) in enumerate(items[:2]):
        cps = in_rows(row0, nrows, bufs[i % 2][0])
        for c in cps:
            c.start()
        inflight[i] = cps
    for i, (row0, nrows) in enumerate(items):
        for c in inflight.pop(i):
            c.wait()
        if i - 2 >= 0:
            pass
        buf, tails = bufs[i % 2]
        if i + 2 < len(items):
            pass
        pending_outs[i] = outs(buf, tails, nrows, row0)
        nxt = i + 2
        if nxt < len(items):
            for c in pending_outs.pop(i):
                c.wait()
            r0, nr = items[nxt]
            cps = in_rows(r0, nr, bufs[nxt % 2][0])
            for c in cps:
                c.start()
            inflight[nxt] = cps
    for cps in pending_outs.values():
        for c in cps:
            c.wait()


def _detile(tables, f_lo, f_hi):
    nf = f_hi - f_lo
    return pl.pallas_call(
        functools.partial(_detile_body, f_lo, f_hi),
        in_specs=[pl.BlockSpec(memory_space=pl.ANY)],
        out_specs=pl.BlockSpec(memory_space=pl.ANY),
        out_shape=jax.ShapeDtypeStruct((nf * VP,), jnp.float32),
        scratch_shapes=[
            pltpu.VMEM((8, V), jnp.float32),
            pltpu.VMEM((8, V), jnp.float32),
            pltpu.VMEM((8, 128), jnp.float32),
            pltpu.VMEM((8, 128), jnp.float32),
            pltpu.SemaphoreType.DMA,
            pltpu.SemaphoreType.DMA,
            pltpu.SemaphoreType.DMA,
        ],
        compiler_params=pltpu.CompilerParams(
            vmem_limit_bytes=100 * 1024 * 1024),
    )(tables)


def _sc_body(f_lo, f_hi, x_hbm, tab_hbm, out_hbm, xrows, idx_v, g_v, o_v, sem):
    nf = f_hi - f_lo
    wid = lax.axis_index("s") * NC + lax.axis_index("c")
    base = wid * RPW

    pltpu.sync_copy(x_hbm.at[pl.ds(base, RPW)], xrows)
    lane = lax.iota(jnp.int32, L)

    def tr_body(j, _):
        rows = lane + j * L
        for fo in range(nf):
            idx_v[pl.ds(fo * RPW + j * L, L)] = plsc.load_gather(
                xrows, [rows, jnp.full((L,), f_lo + fo, jnp.int32)]) + fo * VP
        return 0

    lax.fori_loop(0, RPW // L, tr_body, 0)

    pltpu.async_copy(tab_hbm.at[idx_v], g_v, sem).wait()

    def red_body(j, _):
        acc = g_v[pl.ds(j * L, L)]
        for fo in range(1, nf):
            acc = acc + g_v[pl.ds(fo * RPW + j * L, L)]
        o_v[pl.ds(j * L, L)] = acc
        return 0

    lax.fori_loop(0, RPW // L, red_body, 0)

    pltpu.sync_copy(o_v, out_hbm.at[pl.ds(base, RPW)])


def _sc_chunk(X, tab_flat, f_lo, f_hi):
    nf = f_hi - f_lo
    run = functools.partial(
        pl.kernel,
        out_type=jax.ShapeDtypeStruct((B,), jnp.float32),
        mesh=plsc.VectorSubcoreMesh(core_axis_name="c", subcore_axis_name="s"),
        scratch_types=[
            pltpu.VMEM((RPW, F), jnp.int32),
            pltpu.VMEM((nf * RPW,), jnp.int32),
            pltpu.VMEM((nf * RPW,), jnp.float32),
            pltpu.VMEM((RPW,), jnp.float32),
            pltpu.SemaphoreType.DMA,
        ],
        compiler_params=pltpu.CompilerParams(needs_layout_passes=False),
    )(functools.partial(_sc_body, f_lo, f_hi))
    return run(X, tab_flat)


@jax.jit
def kernel(X, tables):
    partials = []
    for f_lo, f_hi in CHUNKS:
        flat = _detile(tables, f_lo, f_hi)
        partials.append(_sc_chunk(X, flat, f_lo, f_hi))
    out = partials[0] + partials[1]
    return out.reshape(B, 1)


# speedup vs baseline: 1.0124x; 1.0007x over previous
import functools

import jax
import jax.numpy as jnp
from jax import lax
from jax.experimental import pallas as pl
from jax.experimental.pallas import tpu as pltpu
from jax.experimental.pallas import tpu_sc as plsc

B = 16384
F = 26
V = 1000000

NC, NS, L = 2, 16, 16
NW = NC * NS
RPW = B // NW
VP = 1000064

VMAIN = (V // 128) * 128
VTAIL = V - VMAIN

CHUNKS = ((0, 16), (16, 26))


def _detile_body(f_lo, f_hi, tab_hbm, out_hbm, buf_a, buf_b, tails_a, tails_b,
                 sem_in, sem_out, sem_tail):
    nf = f_hi - f_lo
    full_groups = [(f_lo + 8 * g, 8) for g in range(nf // 8)]
    rest = nf % 8

    def in_rows(row0, nrows, buf):
        if nrows == 8:
            return [pltpu.make_async_copy(
                tab_hbm.at[pl.ds(row0, 8)], buf, sem_in)]
        return [pltpu.make_async_copy(tab_hbm.at[row0 + r], buf.at[r], sem_in)
                for r in range(nrows)]

    def outs(buf, tails, nrows, f0):
        cps = []
        for r in range(nrows):
            fo = f0 - f_lo + r
            tails[r, pl.ds(0, VTAIL)] = buf[r, pl.ds(VMAIN, VTAIL)]
            cps.append(pltpu.make_async_copy(
                buf.at[r, pl.ds(0, VMAIN)],
                out_hbm.at[pl.ds(fo * VP, VMAIN)], sem_out))
            cps.append(pltpu.make_async_copy(
                tails.at[r], out_hbm.at[pl.ds(fo * VP + VMAIN, 128)],
                sem_tail))
        for c in cps:
            c.start()
        return cps

    items = full_groups + ([(f_lo + 8 * (nf // 8), rest)] if rest else [])
    bufs = [(buf_a, tails_a), (buf_b, tails_b)]
    inflight = {}
    pending_outs = {}
    for i, (row0, nrows) in enumerate(items[:2]):
        cps = in_rows(row0, nrows, bufs[i % 2][0])
        for c in cps:
            c.start()
        inflight[i] = cps
    for i, (row0, nrows) in enumerate(items):
        for c in inflight.pop(i):
            c.wait()
        buf, tails = bufs[i % 2]
        pending_outs[i] = outs(buf, tails, nrows, row0)
        nxt = i + 2
        if nxt < len(items):
            for c in pending_outs.pop(i):
                c.wait()
            r0, nr = items[nxt]
            cps = in_rows(r0, nr, bufs[nxt % 2][0])
            for c in cps:
                c.start()
            inflight[nxt] = cps
    for cps in pending_outs.values():
        for c in cps:
            c.wait()


def _detile(tables, f_lo, f_hi):
    nf = f_hi - f_lo
    return pl.pallas_call(
        functools.partial(_detile_body, f_lo, f_hi),
        in_specs=[pl.BlockSpec(memory_space=pl.ANY)],
        out_specs=pl.BlockSpec(memory_space=pl.ANY),
        out_shape=jax.ShapeDtypeStruct((nf * VP,), jnp.float32),
        scratch_shapes=[
            pltpu.VMEM((8, V), jnp.float32),
            pltpu.VMEM((8, V), jnp.float32),
            pltpu.VMEM((8, 128), jnp.float32),
            pltpu.VMEM((8, 128), jnp.float32),
            pltpu.SemaphoreType.DMA,
            pltpu.SemaphoreType.DMA,
            pltpu.SemaphoreType.DMA,
        ],
        compiler_params=pltpu.CompilerParams(
            vmem_limit_bytes=100 * 1024 * 1024),
    )(tables)


def _sc_body(f_lo, f_hi, x_hbm, tab_hbm, out_hbm, xrows, idx_v, g_v, o_v, sem):
    nf = f_hi - f_lo
    wid = lax.axis_index("s") * NC + lax.axis_index("c")
    base = wid * RPW

    pltpu.sync_copy(x_hbm.at[pl.ds(base, RPW)], xrows)
    lane = lax.iota(jnp.int32, L)

    def tr_body(j, _):
        rows = lane + j * L
        for fo in range(nf):
            idx_v[pl.ds(fo * RPW + j * L, L)] = plsc.load_gather(
                xrows, [rows, jnp.full((L,), f_lo + fo, jnp.int32)]) + fo * VP
        return 0

    lax.fori_loop(0, RPW // L, tr_body, 0)

    pltpu.async_copy(tab_hbm.at[idx_v], g_v, sem).wait()

    def red_body(j, _):
        acc = g_v[pl.ds(j * L, L)]
        for fo in range(1, nf):
            acc = acc + g_v[pl.ds(fo * RPW + j * L, L)]
        o_v[pl.ds(j * L, L)] = acc
        return 0

    lax.fori_loop(0, RPW // L, red_body, 0)

    pltpu.sync_copy(o_v, out_hbm.at[pl.ds(base, RPW)])


def _sc_chunk(X, tab_flat, f_lo, f_hi):
    nf = f_hi - f_lo
    run = functools.partial(
        pl.kernel,
        out_type=jax.ShapeDtypeStruct((B,), jnp.float32),
        mesh=plsc.VectorSubcoreMesh(core_axis_name="c", subcore_axis_name="s"),
        scratch_types=[
            pltpu.VMEM((RPW, F), jnp.int32),
            pltpu.VMEM((nf * RPW,), jnp.int32),
            pltpu.VMEM((nf * RPW,), jnp.float32),
            pltpu.VMEM((RPW,), jnp.float32),
            pltpu.SemaphoreType.DMA,
        ],
        compiler_params=pltpu.CompilerParams(needs_layout_passes=False),
    )(functools.partial(_sc_body, f_lo, f_hi))
    return run(X, tab_flat)


@jax.jit
def kernel(X, tables):
    partials = []
    for f_lo, f_hi in CHUNKS:
        flat = _detile(tables, f_lo, f_hi)
        partials.append(_sc_chunk(X, flat, f_lo, f_hi))
    out = partials[0] + partials[1]
    return out.reshape(B, 1)
